# baseline (speedup 1.0000x reference)
import functools

import jax
import jax.numpy as jnp
from jax import lax
from jax.experimental import pallas as pl
from jax.experimental.pallas import tpu as pltpu
from jax.experimental.pallas import tpu_sc as plsc

D = 150
DP = 160
N = 10000
E = 160000
NSUB = 16
EPAD = 163840
EPS = EPAD // NSUB
CH = 80
NCHUNK = EPS // CH
STG = 32
NSTG = NCHUNK // STG
NPAD = 10240
RPS = NPAD // NSUB
BLK = 400
NBLK = N // BLK


def _leaky(x):
    return jnp.where(x >= 0, x, 0.01 * x)


@functools.cache
def _sc_segsum_fn():
    mesh = plsc.VectorSubcoreMesh(core_axis_name="c", subcore_axis_name="s")
    return pl.kernel(
        _sc_segsum_body,
        out_type=jax.ShapeDtypeStruct((2, NPAD, DP), jnp.float32),
        mesh=mesh,
        scratch_types=[
            pltpu.VMEM((STG, CH), jnp.int32),
            pltpu.VMEM((STG, CH), jnp.int32),
            pltpu.VMEM((CH, DP), jnp.float32),
            pltpu.VMEM_SHARED((NPAD, DP), jnp.float32),
            pltpu.SemaphoreType.DMA,
        ],
        compiler_params=pltpu.CompilerParams(use_tc_tiling_on_sc=False),
    )


def _sc_segsum(h, srcs, dsts):
    return _sc_segsum_fn()(h, srcs, dsts)


def _sc_segsum_body(h_hbm, src_hbm, dst_hbm, out_hbm,
                    src_v, dst_v, rows_v, acc_sh, sem):
    c = lax.axis_index("c")
    s = lax.axis_index("s")

    zf = jnp.zeros((16,), jnp.float32)

    def _zrow(i, carry):
        for t in range(DP // 16):
            rows_v[i, pl.ds(t * 16, 16)] = zf
        return carry

    lax.fori_loop(0, CH, _zrow, 0)
    for t in range(RPS // CH):
        pltpu.sync_copy(rows_v, acc_sh.at[pl.ds(s * RPS + t * CH, CH)])
    plsc.subcore_barrier()

    def _chunk(k, carry):
        pltpu.async_copy(h_hbm.at[src_v.at[k]], rows_v, sem).wait()
        pltpu.sync_copy(rows_v, acc_sh.at[dst_v.at[k]], add=True)
        return carry

    for st in range(NSTG):
        pltpu.sync_copy(src_hbm.at[c, s, pl.ds(st * STG, STG)], src_v)
        pltpu.sync_copy(dst_hbm.at[c, s, pl.ds(st * STG, STG)], dst_v)
        lax.fori_loop(0, STG, _chunk, 0)
    plsc.subcore_barrier()

    pltpu.sync_copy(acc_sh.at[pl.ds(s * RPS, RPS)],
                    out_hbm.at[c, pl.ds(s * RPS, RPS)])


def _tc_pass_body(a0_r, a1_r, h_r, w0_r, w1_r, wih_r, whh_r, bih_r, bhh_r,
                  hout_r, gacc_r):
    i = pl.program_id(0)
    hb = h_r[...]
    inc = (jnp.dot(a0_r[...], w0_r[...], preferred_element_type=jnp.float32, precision=lax.Precision.HIGHEST)
           + jnp.dot(a1_r[...], w1_r[...], preferred_element_type=jnp.float32, precision=lax.Precision.HIGHEST))
    gi = jnp.dot(inc, wih_r[...], preferred_element_type=jnp.float32) + bih_r[...]
    gh = jnp.dot(hb, whh_r[...], preferred_element_type=jnp.float32) + bhh_r[...]
    r = jax.nn.sigmoid(gi[:, 0:DP] + gh[:, 0:DP])
    z = jax.nn.sigmoid(gi[:, DP:2 * DP] + gh[:, DP:2 * DP])
    n = jnp.tanh(gi[:, 2 * DP:3 * DP] + r * gh[:, 2 * DP:3 * DP])
    hn = (1.0 - z) * n + z * hb
    col = lax.broadcasted_iota(jnp.int32, hn.shape, 1)
    hn = jnp.where(col < D, hn, jnp.where(col == D, 1.0, 0.0))
    hout_r[...] = hn

    @pl.when(i == 0)
    def _():
        gacc_r[...] = jnp.zeros_like(gacc_r)

    gacc_r[...] += jnp.sum(hn, axis=0, keepdims=True)


def _tc_pass(a0, a1, h, w0t, w1t, wiht, whht, bih, bhh):
    full = lambda shape: pl.BlockSpec(shape, lambda i: (0, 0))
    row = lambda shape: pl.BlockSpec(shape, lambda i: (i, 0))
    return pl.pallas_call(
        _tc_pass_body,
        grid=(NBLK,),
        in_specs=[row((BLK, DP)), row((BLK, DP)), row((BLK, DP)),
                  full((DP, DP)), full((DP, DP)),
                  full((DP, 3 * DP)), full((DP, 3 * DP)),
                  full((1, 3 * DP)), full((1, 3 * DP))],
        out_specs=[row((BLK, DP)), full((1, DP))],
        out_shape=[jax.ShapeDtypeStruct((N, DP), jnp.float32),
                   jax.ShapeDtypeStruct((1, DP), jnp.float32)],
    )(a0, a1, h, w0t, w1t, wiht, whht, bih, bhh)


def _head_body(g_r, pt_r, w1_r, b1_r, w2_r, b2_r, wl_r, bl_r, out_r):
    g = g_r[...]
    col = lax.broadcasted_iota(jnp.int32, g.shape, 1)
    pos = (g > 0.0) & (col < D)
    glog = jnp.where(pos, jnp.log(jnp.where(pos, g, 1.0)), 0.0)
    glog = jnp.maximum(glog, 0.0)
    x = jnp.where(col == D, pt_r[0, 0], glog)
    x = _leaky(jnp.dot(x, w1_r[...], preferred_element_type=jnp.float32) + b1_r[...])
    x = _leaky(jnp.dot(x, w2_r[...], preferred_element_type=jnp.float32) + b2_r[...])
    out_r[...] = jnp.dot(x, wl_r[...], preferred_element_type=jnp.float32) + bl_r[...]


def _head(gacc, pt, w1t, b1, w2t, b2, wlt, bl):
    full = lambda shape: pl.BlockSpec(shape, lambda: (0, 0))
    return pl.pallas_call(
        _head_body,
        in_specs=[full((1, DP)),
                  pl.BlockSpec(memory_space=pltpu.SMEM),
                  full((DP, 128)), full((1, 128)),
                  full((128, 128)), full((1, 128)),
                  full((128, 128)), full((1, 128))],
        out_specs=full((1, 128)),
        out_shape=jax.ShapeDtypeStruct((1, 128), jnp.float32),
    )(gacc, pt, w1t, b1, w2t, b2, wlt, bl)


def kernel(nodes, edge_set0, edge_set1, problem_type, tbptt,
           e0_W0, e0_b0, e0_W1, e0_b1,
           e1_W0, e1_b0, e1_W1, e1_b1,
           gru_Wih, gru_Whh, gru_bih, gru_bhh,
           fc1_W, fc1_b, fc2_W, fc2_b, fcL_W, fcL_b):
    f32 = jnp.float32

    h = jnp.concatenate(
        [nodes, jnp.ones((N, 1), f32), jnp.zeros((N, DP - D - 1), f32)], axis=1)

    pad_src = jnp.zeros((EPAD - E,), jnp.int32)
    pad_dst = N + (jnp.arange(EPAD - E, dtype=jnp.int32) % (NPAD - N))
    srcs = jnp.stack([jnp.concatenate([edge_set0[:, 1], pad_src]),
                      jnp.concatenate([edge_set1[:, 1], pad_src])])
    dsts = jnp.stack([jnp.concatenate([edge_set0[:, 0], pad_dst]),
                      jnp.concatenate([edge_set1[:, 0], pad_dst])])
    srcs = srcs.reshape(2, NSUB, NCHUNK, CH)
    dsts = dsts.reshape(2, NSUB, NCHUNK, CH)

    def edge_wt(W, b):
        Wr = W.astype(jnp.bfloat16).astype(f32)
        wt = jnp.zeros((DP, DP), f32)
        wt = wt.at[:D, :D].set(Wr.T)
        wt = wt.at[D, :D].set(b)
        return wt

    w0t = edge_wt(e0_W1, e0_b1)
    w1t = edge_wt(e1_W1, e1_b1)

    def gate_wt(W, b):
        wt = jnp.zeros((DP, 3 * DP), f32)
        bt = jnp.zeros((1, 3 * DP), f32)
        for g in range(3):
            wt = wt.at[:D, g * DP:g * DP + D].set(W[g * D:(g + 1) * D, :].T)
            bt = bt.at[0, g * DP:g * DP + D].set(b[g * D:(g + 1) * D])
        return wt, bt

    wiht, bih = gate_wt(gru_Wih, gru_bih)
    whht, bhh = gate_wt(gru_Whh, gru_bhh)

    fc1t = jnp.zeros((DP, 128), f32).at[:D + 1, :80].set(fc1_W.T)
    b1 = jnp.zeros((1, 128), f32).at[0, :80].set(fc1_b)
    fc2t = jnp.zeros((128, 128), f32).at[:80, :80].set(fc2_W.T)
    b2 = jnp.zeros((1, 128), f32).at[0, :80].set(fc2_b)
    fclt = jnp.zeros((128, 128), f32).at[:80, :10].set(fcL_W.T)
    bl = jnp.zeros((1, 128), f32).at[0, :10].set(fcL_b)

    gacc = None
    for _ in range(2):
        h_sc = h.astype(jnp.bfloat16).astype(f32)
        a = _sc_segsum(h_sc, srcs, dsts)
        h, gacc = _tc_pass(a[0], a[1], h, w0t, w1t, wiht, whht, bih, bhh)

    out = _head(gacc, problem_type, fc1t, b1, fc2t, b2, fclt, bl)
    return out[:, :10]


# speedup vs baseline: 1.6431x; 1.6431x over previous
import functools

import jax
import jax.numpy as jnp
from jax import lax
from jax.experimental import pallas as pl
from jax.experimental.pallas import tpu as pltpu
from jax.experimental.pallas import tpu_sc as plsc

D = 150
DP = 160
N = 10000
E = 160000
NSUB = 16
EPS = E // NSUB
CH = 80
NCHUNK = EPS // CH
STG = 25
NSTG = NCHUNK // STG
NPAD = 10240
RPS = NPAD // NSUB
BLK = 400
NBLK = N // BLK


def _leaky(x):
    return jnp.where(x >= 0, x, 0.01 * x)


@functools.cache
def _sc_segsum_fn():
    mesh = plsc.VectorSubcoreMesh(core_axis_name="c", subcore_axis_name="s")
    return pl.kernel(
        _sc_segsum_body,
        out_type=jax.ShapeDtypeStruct((2, NPAD, DP), jnp.float32),
        mesh=mesh,
        scratch_types=[
            pltpu.VMEM((STG, CH), jnp.int32),
            pltpu.VMEM((STG, CH), jnp.int32),
            pltpu.VMEM((CH, DP), jnp.float32),
            pltpu.VMEM_SHARED((NPAD, DP), jnp.float32),
            pltpu.SemaphoreType.DMA,
        ],
        compiler_params=pltpu.CompilerParams(use_tc_tiling_on_sc=False),
    )


def _sc_segsum(h, srcs, dsts):
    return _sc_segsum_fn()(h, srcs, dsts)


def _sc_segsum_body(h_hbm, src_hbm, dst_hbm, out_hbm,
                    src_v, dst_v, rows_v, acc_sh, sem):
    c = lax.axis_index("c")
    s = lax.axis_index("s")

    zf = jnp.zeros((16,), jnp.float32)

    def _zrow(i, carry):
        for t in range(DP // 16):
            rows_v[i, pl.ds(t * 16, 16)] = zf
        return carry

    lax.fori_loop(0, CH, _zrow, 0)
    for t in range(RPS // CH):
        pltpu.sync_copy(rows_v, acc_sh.at[pl.ds(s * RPS + t * CH, CH)])
    plsc.subcore_barrier()

    def _chunk(k, carry):
        pltpu.async_copy(h_hbm.at[src_v.at[k]], rows_v, sem).wait()
        pltpu.sync_copy(rows_v, acc_sh.at[dst_v.at[k]], add=True)
        return carry

    for st in range(NSTG):
        pltpu.sync_copy(src_hbm.at[c, s, pl.ds(st * STG, STG)], src_v)
        pltpu.sync_copy(dst_hbm.at[c, s, pl.ds(st * STG, STG)], dst_v)
        lax.fori_loop(0, STG, _chunk, 0)
    plsc.subcore_barrier()

    pltpu.sync_copy(acc_sh.at[pl.ds(s * RPS, RPS)],
                    out_hbm.at[c, pl.ds(s * RPS, RPS)])


def _tc_pass_body(a0_r, a1_r, h_r, w0_r, w1_r, wih_r, whh_r, bih_r, bhh_r,
                  hout_r, gacc_r):
    i = pl.program_id(0)
    hb = h_r[...]
    inc = (jnp.dot(a0_r[...], w0_r[...], preferred_element_type=jnp.float32, precision=lax.Precision.HIGHEST)
           + jnp.dot(a1_r[...], w1_r[...], preferred_element_type=jnp.float32, precision=lax.Precision.HIGHEST))
    gi = jnp.dot(inc, wih_r[...], preferred_element_type=jnp.float32) + bih_r[...]
    gh = jnp.dot(hb, whh_r[...], preferred_element_type=jnp.float32) + bhh_r[...]
    r = jax.nn.sigmoid(gi[:, 0:DP] + gh[:, 0:DP])
    z = jax.nn.sigmoid(gi[:, DP:2 * DP] + gh[:, DP:2 * DP])
    n = jnp.tanh(gi[:, 2 * DP:3 * DP] + r * gh[:, 2 * DP:3 * DP])
    hn = (1.0 - z) * n + z * hb
    col = lax.broadcasted_iota(jnp.int32, hn.shape, 1)
    hn = jnp.where(col < D, hn, jnp.where(col == D, 1.0, 0.0))
    hout_r[...] = hn

    @pl.when(i == 0)
    def _():
        gacc_r[...] = jnp.zeros_like(gacc_r)

    gacc_r[...] += jnp.sum(hn, axis=0, keepdims=True)


def _tc_pass(a0, a1, h, w0t, w1t, wiht, whht, bih, bhh):
    full = lambda shape: pl.BlockSpec(shape, lambda i: (0, 0))
    row = lambda shape: pl.BlockSpec(shape, lambda i: (i, 0))
    return pl.pallas_call(
        _tc_pass_body,
        grid=(NBLK,),
        in_specs=[row((BLK, DP)), row((BLK, DP)), row((BLK, DP)),
                  full((DP, DP)), full((DP, DP)),
                  full((DP, 3 * DP)), full((DP, 3 * DP)),
                  full((1, 3 * DP)), full((1, 3 * DP))],
        out_specs=[row((BLK, DP)), full((1, DP))],
        out_shape=[jax.ShapeDtypeStruct((N, DP), jnp.float32),
                   jax.ShapeDtypeStruct((1, DP), jnp.float32)],
    )(a0, a1, h, w0t, w1t, wiht, whht, bih, bhh)


def _head_body(g_r, pt_r, w1_r, b1_r, w2_r, b2_r, wl_r, bl_r, out_r):
    g = g_r[...]
    col = lax.broadcasted_iota(jnp.int32, g.shape, 1)
    pos = (g > 0.0) & (col < D)
    glog = jnp.where(pos, jnp.log(jnp.where(pos, g, 1.0)), 0.0)
    glog = jnp.maximum(glog, 0.0)
    x = jnp.where(col == D, pt_r[0, 0], glog)
    x = _leaky(jnp.dot(x, w1_r[...], preferred_element_type=jnp.float32) + b1_r[...])
    x = _leaky(jnp.dot(x, w2_r[...], preferred_element_type=jnp.float32) + b2_r[...])
    out_r[...] = jnp.dot(x, wl_r[...], preferred_element_type=jnp.float32) + bl_r[...]


def _head(gacc, pt, w1t, b1, w2t, b2, wlt, bl):
    full = lambda shape: pl.BlockSpec(shape, lambda: (0, 0))
    return pl.pallas_call(
        _head_body,
        in_specs=[full((1, DP)),
                  pl.BlockSpec(memory_space=pltpu.SMEM),
                  full((DP, 128)), full((1, 128)),
                  full((128, 128)), full((1, 128)),
                  full((128, 128)), full((1, 128))],
        out_specs=full((1, 128)),
        out_shape=jax.ShapeDtypeStruct((1, 128), jnp.float32),
    )(gacc, pt, w1t, b1, w2t, b2, wlt, bl)


def kernel(nodes, edge_set0, edge_set1, problem_type, tbptt,
           e0_W0, e0_b0, e0_W1, e0_b1,
           e1_W0, e1_b0, e1_W1, e1_b1,
           gru_Wih, gru_Whh, gru_bih, gru_bhh,
           fc1_W, fc1_b, fc2_W, fc2_b, fcL_W, fcL_b):
    f32 = jnp.float32

    h = jnp.concatenate(
        [nodes, jnp.ones((N, 1), f32), jnp.zeros((N, DP - D - 1), f32)], axis=1)

    srcs = jnp.stack([edge_set0[:, 1], edge_set1[:, 1]]).reshape(2, NSUB, NCHUNK, CH)
    dsts = jnp.stack([edge_set0[:, 0], edge_set1[:, 0]]).reshape(2, NSUB, NCHUNK, CH)

    def edge_wt(W, b):
        Wr = W.astype(jnp.bfloat16).astype(f32)
        wt = jnp.zeros((DP, DP), f32)
        wt = wt.at[:D, :D].set(Wr.T)
        wt = wt.at[D, :D].set(b)
        return wt

    w0t = edge_wt(e0_W1, e0_b1)
    w1t = edge_wt(e1_W1, e1_b1)

    def gate_wt(W, b):
        wt = jnp.zeros((DP, 3 * DP), f32)
        bt = jnp.zeros((1, 3 * DP), f32)
        for g in range(3):
            wt = wt.at[:D, g * DP:g * DP + D].set(W[g * D:(g + 1) * D, :].T)
            bt = bt.at[0, g * DP:g * DP + D].set(b[g * D:(g + 1) * D])
        return wt, bt

    wiht, bih = gate_wt(gru_Wih, gru_bih)
    whht, bhh = gate_wt(gru_Whh, gru_bhh)

    fc1t = jnp.zeros((DP, 128), f32).at[:D + 1, :80].set(fc1_W.T)
    b1 = jnp.zeros((1, 128), f32).at[0, :80].set(fc1_b)
    fc2t = jnp.zeros((128, 128), f32).at[:80, :80].set(fc2_W.T)
    b2 = jnp.zeros((1, 128), f32).at[0, :80].set(fc2_b)
    fclt = jnp.zeros((128, 128), f32).at[:80, :10].set(fcL_W.T)
    bl = jnp.zeros((1, 128), f32).at[0, :10].set(fcL_b)

    gacc = None
    for _ in range(2):
        h_sc = h.astype(jnp.bfloat16).astype(f32)
        a = _sc_segsum(h_sc, srcs, dsts)
        h, gacc = _tc_pass(a[0], a[1], h, w0t, w1t, wiht, whht, bih, bhh)

    out = _head(gacc, problem_type, fc1t, b1, fc2t, b2, fclt, bl)
    return out[:, :10]


# speedup vs baseline: 1.7899x; 1.0893x over previous
import functools

import jax
import jax.numpy as jnp
from jax import lax
from jax.experimental import pallas as pl
from jax.experimental.pallas import tpu as pltpu
from jax.experimental.pallas import tpu_sc as plsc

D = 150
DP = 160
N = 10000
E = 160000
NSUB = 16
EPS = E // NSUB
CH = 80
NCHUNK = EPS // CH
STG = 25
NSTG = NCHUNK // STG
NPAD = 10112
RPS = NPAD // NSUB
BLK = 400
NBLK = N // BLK


def _leaky(x):
    return jnp.where(x >= 0, x, 0.01 * x)


@functools.cache
def _sc_segsum_fn():
    mesh = plsc.VectorSubcoreMesh(core_axis_name="c", subcore_axis_name="s")
    return pl.kernel(
        _sc_segsum_body,
        out_type=jax.ShapeDtypeStruct((2, NPAD, DP), jnp.float32),
        mesh=mesh,
        scratch_types=[
            pltpu.VMEM((STG, CH), jnp.int32),
            pltpu.VMEM((STG, CH), jnp.int32),
            pltpu.VMEM((CH, DP), jnp.float32),
            pltpu.VMEM((CH, DP), jnp.float32),
            pltpu.VMEM_SHARED((NPAD, DP), jnp.float32),
            pltpu.SemaphoreType.DMA,
            pltpu.SemaphoreType.DMA,
            pltpu.SemaphoreType.DMA,
            pltpu.SemaphoreType.DMA,
        ],
        compiler_params=pltpu.CompilerParams(use_tc_tiling_on_sc=False),
    )


def _sc_segsum(h, srcs, dsts):
    return _sc_segsum_fn()(h, srcs, dsts)


def _sc_segsum_body(h_hbm, src_hbm, dst_hbm, out_hbm,
                    src_v, dst_v, rows0_v, rows1_v, acc_sh,
                    gsem0, gsem1, ssem0, ssem1):
    c = lax.axis_index("c")
    s = lax.axis_index("s")

    zf = jnp.zeros((16,), jnp.float32)

    def _zrow(i, carry):
        for t in range(DP // 16):
            rows0_v[i, pl.ds(t * 16, 16)] = zf
        return carry

    lax.fori_loop(0, CH, _zrow, 0)
    for t in range(7):
        pltpu.sync_copy(rows0_v, acc_sh.at[pl.ds(s * RPS + t * CH, CH)])
    pltpu.sync_copy(rows0_v.at[pl.ds(0, 72)],
                    acc_sh.at[pl.ds(s * RPS + 560, 72)])
    plsc.subcore_barrier()

    def _pair(j, carry):
        k = 2 * j
        pltpu.async_copy(h_hbm.at[src_v.at[k]], rows0_v, gsem0).wait()
        s0 = pltpu.async_copy(rows0_v, acc_sh.at[dst_v.at[k]], ssem0,
                              add=True)
        pltpu.async_copy(h_hbm.at[src_v.at[k + 1]], rows1_v, gsem1).wait()
        s1 = pltpu.async_copy(rows1_v, acc_sh.at[dst_v.at[k + 1]], ssem1,
                              add=True)
        s0.wait()
        s1.wait()
        return carry

    for st in range(NSTG):
        pltpu.sync_copy(src_hbm.at[c, s, pl.ds(st * STG, STG)], src_v)
        pltpu.sync_copy(dst_hbm.at[c, s, pl.ds(st * STG, STG)], dst_v)
        lax.fori_loop(0, STG // 2, _pair, 0)
        pltpu.async_copy(h_hbm.at[src_v.at[STG - 1]], rows0_v, gsem0).wait()
        pltpu.sync_copy(rows0_v, acc_sh.at[dst_v.at[STG - 1]], add=True)
    plsc.subcore_barrier()

    pltpu.sync_copy(acc_sh.at[pl.ds(s * RPS, RPS)],
                    out_hbm.at[c, pl.ds(s * RPS, RPS)])


def _tc_pass_body(a0_r, a1_r, h_r, w0_r, w1_r, wih_r, whh_r, bih_r, bhh_r,
                  hout_r, gacc_r):
    i = pl.program_id(0)
    hb = h_r[...]
    inc = (jnp.dot(a0_r[...], w0_r[...], preferred_element_type=jnp.float32, precision=lax.Precision.HIGHEST)
           + jnp.dot(a1_r[...], w1_r[...], preferred_element_type=jnp.float32, precision=lax.Precision.HIGHEST))
    gi = jnp.dot(inc, wih_r[...], preferred_element_type=jnp.float32) + bih_r[...]
    gh = jnp.dot(hb, whh_r[...], preferred_element_type=jnp.float32) + bhh_r[...]
    r = jax.nn.sigmoid(gi[:, 0:DP] + gh[:, 0:DP])
    z = jax.nn.sigmoid(gi[:, DP:2 * DP] + gh[:, DP:2 * DP])
    n = jnp.tanh(gi[:, 2 * DP:3 * DP] + r * gh[:, 2 * DP:3 * DP])
    hn = (1.0 - z) * n + z * hb
    col = lax.broadcasted_iota(jnp.int32, hn.shape, 1)
    hn = jnp.where(col < D, hn, jnp.where(col == D, 1.0, 0.0))
    hout_r[...] = hn

    @pl.when(i == 0)
    def _():
        gacc_r[...] = jnp.zeros_like(gacc_r)

    gacc_r[...] += jnp.sum(hn, axis=0, keepdims=True)


def _tc_pass(a0, a1, h, w0t, w1t, wiht, whht, bih, bhh):
    full = lambda shape: pl.BlockSpec(shape, lambda i: (0, 0))
    row = lambda shape: pl.BlockSpec(shape, lambda i: (i, 0))
    return pl.pallas_call(
        _tc_pass_body,
        grid=(NBLK,),
        in_specs=[row((BLK, DP)), row((BLK, DP)), row((BLK, DP)),
                  full((DP, DP)), full((DP, DP)),
                  full((DP, 3 * DP)), full((DP, 3 * DP)),
                  full((1, 3 * DP)), full((1, 3 * DP))],
        out_specs=[row((BLK, DP)), full((1, DP))],
        out_shape=[jax.ShapeDtypeStruct((N, DP), jnp.float32),
                   jax.ShapeDtypeStruct((1, DP), jnp.float32)],
    )(a0, a1, h, w0t, w1t, wiht, whht, bih, bhh)


def _head_body(g_r, pt_r, w1_r, b1_r, w2_r, b2_r, wl_r, bl_r, out_r):
    g = g_r[...]
    col = lax.broadcasted_iota(jnp.int32, g.shape, 1)
    pos = (g > 0.0) & (col < D)
    glog = jnp.where(pos, jnp.log(jnp.where(pos, g, 1.0)), 0.0)
    glog = jnp.maximum(glog, 0.0)
    x = jnp.where(col == D, pt_r[0, 0], glog)
    x = _leaky(jnp.dot(x, w1_r[...], preferred_element_type=jnp.float32) + b1_r[...])
    x = _leaky(jnp.dot(x, w2_r[...], preferred_element_type=jnp.float32) + b2_r[...])
    out_r[...] = jnp.dot(x, wl_r[...], preferred_element_type=jnp.float32) + bl_r[...]


def _head(gacc, pt, w1t, b1, w2t, b2, wlt, bl):
    full = lambda shape: pl.BlockSpec(shape, lambda: (0, 0))
    return pl.pallas_call(
        _head_body,
        in_specs=[full((1, DP)),
                  pl.BlockSpec(memory_space=pltpu.SMEM),
                  full((DP, 128)), full((1, 128)),
                  full((128, 128)), full((1, 128)),
                  full((128, 128)), full((1, 128))],
        out_specs=full((1, 128)),
        out_shape=jax.ShapeDtypeStruct((1, 128), jnp.float32),
    )(gacc, pt, w1t, b1, w2t, b2, wlt, bl)


def kernel(nodes, edge_set0, edge_set1, problem_type, tbptt,
           e0_W0, e0_b0, e0_W1, e0_b1,
           e1_W0, e1_b0, e1_W1, e1_b1,
           gru_Wih, gru_Whh, gru_bih, gru_bhh,
           fc1_W, fc1_b, fc2_W, fc2_b, fcL_W, fcL_b):
    f32 = jnp.float32

    h = jnp.concatenate(
        [nodes, jnp.ones((N, 1), f32), jnp.zeros((N, DP - D - 1), f32)], axis=1)

    srcs = jnp.stack([edge_set0[:, 1], edge_set1[:, 1]]).reshape(2, NSUB, NCHUNK, CH)
    dsts = jnp.stack([edge_set0[:, 0], edge_set1[:, 0]]).reshape(2, NSUB, NCHUNK, CH)

    def edge_wt(W, b):
        Wr = W.astype(jnp.bfloat16).astype(f32)
        wt = jnp.zeros((DP, DP), f32)
        wt = wt.at[:D, :D].set(Wr.T)
        wt = wt.at[D, :D].set(b)
        return wt

    w0t = edge_wt(e0_W1, e0_b1)
    w1t = edge_wt(e1_W1, e1_b1)

    def gate_wt(W, b):
        wt = jnp.zeros((DP, 3 * DP), f32)
        bt = jnp.zeros((1, 3 * DP), f32)
        for g in range(3):
            wt = wt.at[:D, g * DP:g * DP + D].set(W[g * D:(g + 1) * D, :].T)
            bt = bt.at[0, g * DP:g * DP + D].set(b[g * D:(g + 1) * D])
        return wt, bt

    wiht, bih = gate_wt(gru_Wih, gru_bih)
    whht, bhh = gate_wt(gru_Whh, gru_bhh)

    fc1t = jnp.zeros((DP, 128), f32).at[:D + 1, :80].set(fc1_W.T)
    b1 = jnp.zeros((1, 128), f32).at[0, :80].set(fc1_b)
    fc2t = jnp.zeros((128, 128), f32).at[:80, :80].set(fc2_W.T)
    b2 = jnp.zeros((1, 128), f32).at[0, :80].set(fc2_b)
    fclt = jnp.zeros((128, 128), f32).at[:80, :10].set(fcL_W.T)
    bl = jnp.zeros((1, 128), f32).at[0, :10].set(fcL_b)

    gacc = None
    for _ in range(2):
        h_sc = h.astype(jnp.bfloat16).astype(f32)
        a = _sc_segsum(h_sc, srcs, dsts)
        h, gacc = _tc_pass(a[0], a[1], h, w0t, w1t, wiht, whht, bih, bhh)

    out = _head(gacc, problem_type, fc1t, b1, fc2t, b2, fclt, bl)
    return out[:, :10]


# speedup vs baseline: 1.9437x; 1.0859x over previous
import functools

import jax
import jax.numpy as jnp
from jax import lax
from jax.experimental import pallas as pl
from jax.experimental.pallas import tpu as pltpu
from jax.experimental.pallas import tpu_sc as plsc

D = 150
DP = 160
N = 10000
E = 160000
NSUB = 16
EPS = E // NSUB
CH = 80
NCHUNK = EPS // CH
STG = 25
NSTG = NCHUNK // STG
NPAD = 10112
RPS = NPAD // NSUB
BLK = 400
NBLK = N // BLK


def _leaky(x):
    return jnp.where(x >= 0, x, 0.01 * x)


@functools.cache
def _sc_segsum_fn():
    mesh = plsc.VectorSubcoreMesh(core_axis_name="c", subcore_axis_name="s")
    return pl.kernel(
        _sc_segsum_body,
        out_type=jax.ShapeDtypeStruct((2, NPAD, DP), jnp.float32),
        mesh=mesh,
        scratch_types=[
            pltpu.VMEM((STG, CH), jnp.int32),
            pltpu.VMEM((STG, CH), jnp.int32),
            pltpu.VMEM((CH, DP), jnp.float32),
            pltpu.VMEM((CH, DP), jnp.float32),
            pltpu.VMEM_SHARED((NPAD, DP), jnp.float32),
            pltpu.SemaphoreType.DMA,
            pltpu.SemaphoreType.DMA,
            pltpu.SemaphoreType.DMA,
            pltpu.SemaphoreType.DMA,
        ],
        compiler_params=pltpu.CompilerParams(use_tc_tiling_on_sc=False),
    )


def _sc_segsum(h, srcs, dsts):
    return _sc_segsum_fn()(h, srcs, dsts)


def _sc_segsum_body(h_hbm, src_hbm, dst_hbm, out_hbm,
                    src_v, dst_v, rows0_v, rows1_v, acc_sh,
                    gsem0, gsem1, ssem0, ssem1):
    c = lax.axis_index("c")
    s = lax.axis_index("s")

    zf = jnp.zeros((16,), jnp.float32)

    def _zrow(i, carry):
        for t in range(DP // 16):
            rows0_v[i, pl.ds(t * 16, 16)] = zf
        return carry

    lax.fori_loop(0, CH, _zrow, 0)
    for t in range(7):
        pltpu.sync_copy(rows0_v, acc_sh.at[pl.ds(s * RPS + t * CH, CH)])
    pltpu.sync_copy(rows0_v.at[pl.ds(0, 72)],
                    acc_sh.at[pl.ds(s * RPS + 560, 72)])
    plsc.subcore_barrier()

    rows = (rows0_v, rows1_v)
    gsem = (gsem0, gsem1)
    ssem = (ssem0, ssem1)
    for st in range(NSTG):
        pltpu.sync_copy(src_hbm.at[c, s, pl.ds(st * STG, STG)], src_v)
        pltpu.sync_copy(dst_hbm.at[c, s, pl.ds(st * STG, STG)], dst_v)
        pend_g = [None, None]
        pend_s = [None, None]
        pend_g[0] = pltpu.async_copy(h_hbm.at[src_v.at[0]], rows0_v, gsem0)
        for k in range(STG):
            p = k & 1
            q = 1 - p
            pend_g[p].wait()
            if k + 1 < STG:
                if pend_s[q] is not None:
                    pend_s[q].wait()
                pend_g[q] = pltpu.async_copy(h_hbm.at[src_v.at[k + 1]],
                                             rows[q], gsem[q])
            pend_s[p] = pltpu.async_copy(rows[p], acc_sh.at[dst_v.at[k]],
                                         ssem[p], add=True)
        pend_s[0].wait()
        pend_s[1].wait()
    plsc.subcore_barrier()

    pltpu.sync_copy(acc_sh.at[pl.ds(s * RPS, RPS)],
                    out_hbm.at[c, pl.ds(s * RPS, RPS)])


def _tc_pass_body(a0_r, a1_r, h_r, w0_r, w1_r, wih_r, whh_r, bih_r, bhh_r,
                  hout_r, gacc_r):
    i = pl.program_id(0)
    hb = h_r[...]
    inc = (jnp.dot(a0_r[...], w0_r[...], preferred_element_type=jnp.float32, precision=lax.Precision.HIGHEST)
           + jnp.dot(a1_r[...], w1_r[...], preferred_element_type=jnp.float32, precision=lax.Precision.HIGHEST))
    gi = jnp.dot(inc, wih_r[...], preferred_element_type=jnp.float32) + bih_r[...]
    gh = jnp.dot(hb, whh_r[...], preferred_element_type=jnp.float32) + bhh_r[...]
    r = jax.nn.sigmoid(gi[:, 0:DP] + gh[:, 0:DP])
    z = jax.nn.sigmoid(gi[:, DP:2 * DP] + gh[:, DP:2 * DP])
    n = jnp.tanh(gi[:, 2 * DP:3 * DP] + r * gh[:, 2 * DP:3 * DP])
    hn = (1.0 - z) * n + z * hb
    col = lax.broadcasted_iota(jnp.int32, hn.shape, 1)
    hn = jnp.where(col < D, hn, jnp.where(col == D, 1.0, 0.0))
    hout_r[...] = hn

    @pl.when(i == 0)
    def _():
        gacc_r[...] = jnp.zeros_like(gacc_r)

    gacc_r[...] += jnp.sum(hn, axis=0, keepdims=True)


def _tc_pass(a0, a1, h, w0t, w1t, wiht, whht, bih, bhh):
    full = lambda shape: pl.BlockSpec(shape, lambda i: (0, 0))
    row = lambda shape: pl.BlockSpec(shape, lambda i: (i, 0))
    return pl.pallas_call(
        _tc_pass_body,
        grid=(NBLK,),
        in_specs=[row((BLK, DP)), row((BLK, DP)), row((BLK, DP)),
                  full((DP, DP)), full((DP, DP)),
                  full((DP, 3 * DP)), full((DP, 3 * DP)),
                  full((1, 3 * DP)), full((1, 3 * DP))],
        out_specs=[row((BLK, DP)), full((1, DP))],
        out_shape=[jax.ShapeDtypeStruct((N, DP), jnp.float32),
                   jax.ShapeDtypeStruct((1, DP), jnp.float32)],
    )(a0, a1, h, w0t, w1t, wiht, whht, bih, bhh)


def _head_body(g_r, pt_r, w1_r, b1_r, w2_r, b2_r, wl_r, bl_r, out_r):
    g = g_r[...]
    col = lax.broadcasted_iota(jnp.int32, g.shape, 1)
    pos = (g > 0.0) & (col < D)
    glog = jnp.where(pos, jnp.log(jnp.where(pos, g, 1.0)), 0.0)
    glog = jnp.maximum(glog, 0.0)
    x = jnp.where(col == D, pt_r[0, 0], glog)
    x = _leaky(jnp.dot(x, w1_r[...], preferred_element_type=jnp.float32) + b1_r[...])
    x = _leaky(jnp.dot(x, w2_r[...], preferred_element_type=jnp.float32) + b2_r[...])
    out_r[...] = jnp.dot(x, wl_r[...], preferred_element_type=jnp.float32) + bl_r[...]


def _head(gacc, pt, w1t, b1, w2t, b2, wlt, bl):
    full = lambda shape: pl.BlockSpec(shape, lambda: (0, 0))
    return pl.pallas_call(
        _head_body,
        in_specs=[full((1, DP)),
                  pl.BlockSpec(memory_space=pltpu.SMEM),
                  full((DP, 128)), full((1, 128)),
                  full((128, 128)), full((1, 128)),
                  full((128, 128)), full((1, 128))],
        out_specs=full((1, 128)),
        out_shape=jax.ShapeDtypeStruct((1, 128), jnp.float32),
    )(gacc, pt, w1t, b1, w2t, b2, wlt, bl)


def kernel(nodes, edge_set0, edge_set1, problem_type, tbptt,
           e0_W0, e0_b0, e0_W1, e0_b1,
           e1_W0, e1_b0, e1_W1, e1_b1,
           gru_Wih, gru_Whh, gru_bih, gru_bhh,
           fc1_W, fc1_b, fc2_W, fc2_b, fcL_W, fcL_b):
    f32 = jnp.float32

    h = jnp.concatenate(
        [nodes, jnp.ones((N, 1), f32), jnp.zeros((N, DP - D - 1), f32)], axis=1)

    srcs = jnp.stack([edge_set0[:, 1], edge_set1[:, 1]]).reshape(2, NSUB, NCHUNK, CH)
    dsts = jnp.stack([edge_set0[:, 0], edge_set1[:, 0]]).reshape(2, NSUB, NCHUNK, CH)

    def edge_wt(W, b):
        Wr = W.astype(jnp.bfloat16).astype(f32)
        wt = jnp.zeros((DP, DP), f32)
        wt = wt.at[:D, :D].set(Wr.T)
        wt = wt.at[D, :D].set(b)
        return wt

    w0t = edge_wt(e0_W1, e0_b1)
    w1t = edge_wt(e1_W1, e1_b1)

    def gate_wt(W, b):
        wt = jnp.zeros((DP, 3 * DP), f32)
        bt = jnp.zeros((1, 3 * DP), f32)
        for g in range(3):
            wt = wt.at[:D, g * DP:g * DP + D].set(W[g * D:(g + 1) * D, :].T)
            bt = bt.at[0, g * DP:g * DP + D].set(b[g * D:(g + 1) * D])
        return wt, bt

    wiht, bih = gate_wt(gru_Wih, gru_bih)
    whht, bhh = gate_wt(gru_Whh, gru_bhh)

    fc1t = jnp.zeros((DP, 128), f32).at[:D + 1, :80].set(fc1_W.T)
    b1 = jnp.zeros((1, 128), f32).at[0, :80].set(fc1_b)
    fc2t = jnp.zeros((128, 128), f32).at[:80, :80].set(fc2_W.T)
    b2 = jnp.zeros((1, 128), f32).at[0, :80].set(fc2_b)
    fclt = jnp.zeros((128, 128), f32).at[:80, :10].set(fcL_W.T)
    bl = jnp.zeros((1, 128), f32).at[0, :10].set(fcL_b)

    gacc = None
    for _ in range(2):
        h_sc = h.astype(jnp.bfloat16).astype(f32)
        a = _sc_segsum(h_sc, srcs, dsts)
        h, gacc = _tc_pass(a[0], a[1], h, w0t, w1t, wiht, whht, bih, bhh)

    out = _head(gacc, problem_type, fc1t, b1, fc2t, b2, fclt, bl)
    return out[:, :10]


# speedup vs baseline: 1.9905x; 1.0241x over previous
import functools

import jax
import jax.numpy as jnp
from jax import lax
from jax.experimental import pallas as pl
from jax.experimental.pallas import tpu as pltpu
from jax.experimental.pallas import tpu_sc as plsc

D = 150
DP = 160
N = 10000
E = 160000
NSUB = 16
EPS = E // NSUB
CH = 80
NCHUNK = EPS // CH
STG = 25
NSTG = NCHUNK // STG
NPAD = 10112
RPS = NPAD // NSUB
BLK = 400
NBLK = N // BLK


def _leaky(x):
    return jnp.where(x >= 0, x, 0.01 * x)


@functools.cache
def _sc_segsum_fn():
    mesh = plsc.VectorSubcoreMesh(core_axis_name="c", subcore_axis_name="s")
    return pl.kernel(
        _sc_segsum_body,
        out_type=jax.ShapeDtypeStruct((2, NPAD, DP), jnp.float32),
        mesh=mesh,
        scratch_types=[
            pltpu.VMEM((STG, CH), jnp.int32),
            pltpu.VMEM((STG, CH), jnp.int32),
            pltpu.VMEM((CH, DP), jnp.float32),
            pltpu.VMEM((CH, DP), jnp.float32),
            pltpu.VMEM_SHARED((NPAD, DP), jnp.float32),
            pltpu.SemaphoreType.DMA,
            pltpu.SemaphoreType.DMA,
            pltpu.SemaphoreType.DMA,
            pltpu.SemaphoreType.DMA,
        ],
        compiler_params=pltpu.CompilerParams(use_tc_tiling_on_sc=False),
    )


def _sc_segsum(h, srcs, dsts):
    return _sc_segsum_fn()(h, srcs, dsts)


def _sc_segsum_body(h_hbm, src_hbm, dst_hbm, out_hbm,
                    src_v, dst_v, rows0_v, rows1_v, acc_sh,
                    gsem0, gsem1, ssem0, ssem1):
    c = lax.axis_index("c")
    s = lax.axis_index("s")

    zf = jnp.zeros((16,), jnp.float32)

    def _zrow(i, carry):
        for t in range(DP // 16):
            rows0_v[i, pl.ds(t * 16, 16)] = zf
        return carry

    lax.fori_loop(0, CH, _zrow, 0)
    for t in range(7):
        pltpu.sync_copy(rows0_v, acc_sh.at[pl.ds(s * RPS + t * CH, CH)])
    pltpu.sync_copy(rows0_v.at[pl.ds(0, 72)],
                    acc_sh.at[pl.ds(s * RPS + 560, 72)])
    plsc.subcore_barrier()

    rows = (rows0_v, rows1_v)
    gsem = (gsem0, gsem1)
    ssem = (ssem0, ssem1)
    for st in range(NSTG):
        pltpu.sync_copy(src_hbm.at[c, s, pl.ds(st * STG, STG)], src_v)
        pltpu.sync_copy(dst_hbm.at[c, s, pl.ds(st * STG, STG)], dst_v)
        pend_g = [None, None]
        pend_s = [None, None]
        pend_g[0] = pltpu.async_copy(h_hbm.at[src_v.at[0]], rows0_v, gsem0)
        for k in range(STG):
            p = k & 1
            q = 1 - p
            pend_g[p].wait()
            if k + 1 < STG:
                if pend_s[q] is not None:
                    pend_s[q].wait()
                pend_g[q] = pltpu.async_copy(h_hbm.at[src_v.at[k + 1]],
                                             rows[q], gsem[q])
            pend_s[p] = pltpu.async_copy(rows[p], acc_sh.at[dst_v.at[k]],
                                         ssem[p], add=True)
        pend_s[0].wait()
        pend_s[1].wait()
    plsc.subcore_barrier()

    pltpu.sync_copy(acc_sh.at[pl.ds(s * RPS, RPS)],
                    out_hbm.at[c, pl.ds(s * RPS, RPS)])


def _tc_gh_body(h_r, whh_r, bhh_r, gh_r):
    gh_r[...] = (jnp.dot(h_r[...], whh_r[...],
                         preferred_element_type=jnp.float32) + bhh_r[...])


def _tc_gh(h, whht, bhh):
    full = lambda shape: pl.BlockSpec(shape, lambda i: (0, 0))
    row = lambda shape: pl.BlockSpec(shape, lambda i: (i, 0))
    return pl.pallas_call(
        _tc_gh_body,
        grid=(NBLK,),
        in_specs=[row((BLK, DP)), full((DP, 3 * DP)), full((1, 3 * DP))],
        out_specs=row((BLK, 3 * DP)),
        out_shape=jax.ShapeDtypeStruct((N, 3 * DP), jnp.float32),
    )(h, whht, bhh)


def _tc_pass_body(a0_r, a1_r, h_r, gh_full_r, w0_r, w1_r, wih_r, bih_r,
                  hout_r, gacc_r):
    i = pl.program_id(0)
    hb = h_r[...]
    inc = (jnp.dot(a0_r[...], w0_r[...], preferred_element_type=jnp.float32, precision=lax.Precision.HIGHEST)
           + jnp.dot(a1_r[...], w1_r[...], preferred_element_type=jnp.float32, precision=lax.Precision.HIGHEST))
    gi = jnp.dot(inc, wih_r[...], preferred_element_type=jnp.float32) + bih_r[...]
    gh = gh_full_r[...]
    r = jax.nn.sigmoid(gi[:, 0:DP] + gh[:, 0:DP])
    z = jax.nn.sigmoid(gi[:, DP:2 * DP] + gh[:, DP:2 * DP])
    n = jnp.tanh(gi[:, 2 * DP:3 * DP] + r * gh[:, 2 * DP:3 * DP])
    hn = (1.0 - z) * n + z * hb
    col = lax.broadcasted_iota(jnp.int32, hn.shape, 1)
    hn = jnp.where(col < D, hn, jnp.where(col == D, 1.0, 0.0))
    hout_r[...] = hn

    @pl.when(i == 0)
    def _():
        gacc_r[...] = jnp.zeros_like(gacc_r)

    gacc_r[...] += jnp.sum(hn, axis=0, keepdims=True)


def _tc_pass(a0, a1, h, gh, w0t, w1t, wiht, bih):
    full = lambda shape: pl.BlockSpec(shape, lambda i: (0, 0))
    row = lambda shape: pl.BlockSpec(shape, lambda i: (i, 0))
    return pl.pallas_call(
        _tc_pass_body,
        grid=(NBLK,),
        in_specs=[row((BLK, DP)), row((BLK, DP)), row((BLK, DP)),
                  row((BLK, 3 * DP)),
                  full((DP, DP)), full((DP, DP)),
                  full((DP, 3 * DP)), full((1, 3 * DP))],
        out_specs=[row((BLK, DP)), full((1, DP))],
        out_shape=[jax.ShapeDtypeStruct((N, DP), jnp.float32),
                   jax.ShapeDtypeStruct((1, DP), jnp.float32)],
    )(a0, a1, h, gh, w0t, w1t, wiht, bih)


def _head_body(g_r, pt_r, w1_r, b1_r, w2_r, b2_r, wl_r, bl_r, out_r):
    g = g_r[...]
    col = lax.broadcasted_iota(jnp.int32, g.shape, 1)
    pos = (g > 0.0) & (col < D)
    glog = jnp.where(pos, jnp.log(jnp.where(pos, g, 1.0)), 0.0)
    glog = jnp.maximum(glog, 0.0)
    x = jnp.where(col == D, pt_r[0, 0], glog)
    x = _leaky(jnp.dot(x, w1_r[...], preferred_element_type=jnp.float32) + b1_r[...])
    x = _leaky(jnp.dot(x, w2_r[...], preferred_element_type=jnp.float32) + b2_r[...])
    out_r[...] = jnp.dot(x, wl_r[...], preferred_element_type=jnp.float32) + bl_r[...]


def _head(gacc, pt, w1t, b1, w2t, b2, wlt, bl):
    full = lambda shape: pl.BlockSpec(shape, lambda: (0, 0))
    return pl.pallas_call(
        _head_body,
        in_specs=[full((1, DP)),
                  pl.BlockSpec(memory_space=pltpu.SMEM),
                  full((DP, 128)), full((1, 128)),
                  full((128, 128)), full((1, 128)),
                  full((128, 128)), full((1, 128))],
        out_specs=full((1, 128)),
        out_shape=jax.ShapeDtypeStruct((1, 128), jnp.float32),
    )(gacc, pt, w1t, b1, w2t, b2, wlt, bl)


def kernel(nodes, edge_set0, edge_set1, problem_type, tbptt,
           e0_W0, e0_b0, e0_W1, e0_b1,
           e1_W0, e1_b0, e1_W1, e1_b1,
           gru_Wih, gru_Whh, gru_bih, gru_bhh,
           fc1_W, fc1_b, fc2_W, fc2_b, fcL_W, fcL_b):
    f32 = jnp.float32

    h = jnp.concatenate(
        [nodes, jnp.ones((N, 1), f32), jnp.zeros((N, DP - D - 1), f32)], axis=1)

    srcs = jnp.stack([edge_set0[:, 1], edge_set1[:, 1]]).reshape(2, NSUB, NCHUNK, CH)
    dsts = jnp.stack([edge_set0[:, 0], edge_set1[:, 0]]).reshape(2, NSUB, NCHUNK, CH)

    def edge_wt(W, b):
        Wr = W.astype(jnp.bfloat16).astype(f32)
        wt = jnp.zeros((DP, DP), f32)
        wt = wt.at[:D, :D].set(Wr.T)
        wt = wt.at[D, :D].set(b)
        return wt

    w0t = edge_wt(e0_W1, e0_b1)
    w1t = edge_wt(e1_W1, e1_b1)

    def gate_wt(W, b):
        wt = jnp.zeros((DP, 3 * DP), f32)
        bt = jnp.zeros((1, 3 * DP), f32)
        for g in range(3):
            wt = wt.at[:D, g * DP:g * DP + D].set(W[g * D:(g + 1) * D, :].T)
            bt = bt.at[0, g * DP:g * DP + D].set(b[g * D:(g + 1) * D])
        return wt, bt

    wiht, bih = gate_wt(gru_Wih, gru_bih)
    whht, bhh = gate_wt(gru_Whh, gru_bhh)

    fc1t = jnp.zeros((DP, 128), f32).at[:D + 1, :80].set(fc1_W.T)
    b1 = jnp.zeros((1, 128), f32).at[0, :80].set(fc1_b)
    fc2t = jnp.zeros((128, 128), f32).at[:80, :80].set(fc2_W.T)
    b2 = jnp.zeros((1, 128), f32).at[0, :80].set(fc2_b)
    fclt = jnp.zeros((128, 128), f32).at[:80, :10].set(fcL_W.T)
    bl = jnp.zeros((1, 128), f32).at[0, :10].set(fcL_b)

    gacc = None
    for _ in range(2):
        h_sc = h.astype(jnp.bfloat16).astype(f32)
        a = _sc_segsum(h_sc, srcs, dsts)
        gh = _tc_gh(h, whht, bhh)
        h, gacc = _tc_pass(a[0], a[1], h, gh, w0t, w1t, wiht, bih)

    out = _head(gacc, problem_type, fc1t, b1, fc2t, b2, fclt, bl)
    return out[:, :10]
